# Initial kernel scaffold; baseline (speedup 1.0000x reference)
import jax
import jax.numpy as jnp
from jax.experimental import pallas as pl


def kernel(x, cell_graph, gene_graph, Wl_c, Wr_c, b_c, Wl_g, Wr_g, b_g):
    raise NotImplementedError("write your pallas kernel here")


# speedup vs baseline: 2.5930x; 2.5930x over previous
import functools

import jax
import jax.numpy as jnp
from jax import lax
from jax.experimental import pallas as pl
from jax.experimental.pallas import tpu as pltpu
from jax.experimental.pallas import tpu_sc as plsc

N_CELLS = 10000
N_GENES = 128
E_CELL = 320000
E_GENE = 2048

_NC = 2
_NS = 16
_NW = _NC * _NS
_CK = 128
_CH = 79
_EPW = _CH * _CK
_EPAD = _NW * _EPW
_NPAD = 10112
_RPS = _NPAD // _NS

@functools.lru_cache(maxsize=1)
def _make_sc_cell_agg():
    mesh = plsc.VectorSubcoreMesh(core_axis_name="c", subcore_axis_name="s")

    @functools.partial(
        pl.kernel,
        mesh=mesh,
        out_type=jax.ShapeDtypeStruct((_NC, _NPAD, 128), jnp.float32),
        scratch_types=[
            pltpu.VMEM((_CH, _CK), jnp.int32),
            pltpu.VMEM((_CH, _CK), jnp.int32),
            pltpu.VMEM((_CK, 128), jnp.float32),
            pltpu.VMEM_SHARED((_NPAD, 128), jnp.float32),
            pltpu.SemaphoreType.DMA,
        ],
    )
    def _sc_cell_agg(xe_hbm, srcp_hbm, dstp_hbm, zeros_hbm, sum_out,
                     src_v, dst_v, rows_v, ssum, sem):
        cid = lax.axis_index("c")
        sid = lax.axis_index("s")
        w = cid * _NS + sid
        sl = pl.ds(sid * _RPS, _RPS)
        pltpu.sync_copy(zeros_hbm.at[sl], ssum.at[sl])
        pltpu.sync_copy(srcp_hbm.at[w], src_v)
        pltpu.sync_copy(dstp_hbm.at[w], dst_v)
        plsc.subcore_barrier()

        def chunk(ci, carry):
            pltpu.async_copy(xe_hbm.at[src_v.at[ci]], rows_v, sem).wait()
            pltpu.sync_copy(rows_v, ssum.at[dst_v.at[ci]], add=True)
            return carry

        lax.fori_loop(0, _CH, chunk, 0)
        plsc.subcore_barrier()
        pltpu.sync_copy(ssum.at[sl], sum_out.at[cid].at[sl])

    return _sc_cell_agg


def _cell_update_body(sum_ref, cnt_ref, x_ref, wl_ref, wr_ref, b_ref, o_ref):
    s = sum_ref[0] + sum_ref[1]
    cnt = cnt_ref[0] + cnt_ref[1]
    r = 1.0 / jnp.clip(cnt[:, 0:1], 1.0, None)
    agg = s * r
    cdims = (((1,), (1,)), ((), ()))
    o_ref[...] = (
        lax.dot_general(agg, wl_ref[...], cdims,
                        preferred_element_type=jnp.float32)
        + lax.dot_general(x_ref[...], wr_ref[...], cdims,
                          preferred_element_type=jnp.float32)
        + b_ref[...]
    )


def _tc_cell_update(sum_parts, cnt_parts, x, wl, wr, b2d):
    br = 1000
    grid = (N_CELLS // br,)
    return pl.pallas_call(
        _cell_update_body,
        grid=grid,
        in_specs=[
            pl.BlockSpec((_NC, br, 128), lambda i: (0, i, 0)),
            pl.BlockSpec((_NC, br, 128), lambda i: (0, i, 0)),
            pl.BlockSpec((br, 128), lambda i: (i, 0)),
            pl.BlockSpec((128, 128), lambda i: (0, 0)),
            pl.BlockSpec((128, 128), lambda i: (0, 0)),
            pl.BlockSpec((1, 128), lambda i: (0, 0)),
        ],
        out_specs=pl.BlockSpec((br, 128), lambda i: (i, 0)),
        out_shape=jax.ShapeDtypeStruct((N_CELLS, 128), jnp.float32),
    )(sum_parts, cnt_parts, x, wl, wr, b2d)


def _gene_m_body(gg_ref, o_ref):
    gg = gg_ref[...]
    src = gg[0:1, :]
    dst = gg[1:2, :]
    ids = lax.broadcasted_iota(jnp.int32, (N_GENES, E_GENE), 0)
    dhot = (ids == dst).astype(jnp.float32)
    shot = (ids == src).astype(jnp.float32)
    m_raw = lax.dot_general(dhot, shot, (((1,), (1,)), ((), ())),
                            preferred_element_type=jnp.float32)
    cnt = jnp.sum(dhot, axis=1, keepdims=True)
    o_ref[...] = m_raw * (1.0 / jnp.clip(cnt, 1.0, None))


def _tc_gene_m(gene_graph):
    return pl.pallas_call(
        _gene_m_body,
        out_shape=jax.ShapeDtypeStruct((N_GENES, N_GENES), jnp.float32),
    )(gene_graph)


_GBC = 256
_GMAIN = (N_CELLS // _GBC) * _GBC
_GTAIL = N_CELLS - _GMAIN


def _gene_layer_body(xt_ref, m_ref, wl_ref, wr_ref, b_ref, o_ref, agg_ref):
    c = pl.program_id(0)

    @pl.when(c == 0)
    def _():
        agg_ref[...] = lax.dot_general(
            m_ref[...], xt_ref[...], (((1,), (0,)), ((), ())),
            preferred_element_type=jnp.float32)

    cdims = (((1,), (1,)), ((), ()))
    o_ref[...] = (
        lax.dot_general(agg_ref[...], wl_ref[...], cdims,
                        preferred_element_type=jnp.float32)
        + lax.dot_general(xt_ref[...], wr_ref[...], cdims,
                          preferred_element_type=jnp.float32)
        + jnp.broadcast_to(b_ref[...], (N_GENES, _GBC))
    )


def _gene_tail_body(xt_ref, agg_ref, wl_ref, wr_ref, b_ref, o_ref):
    cdims = (((1,), (1,)), ((), ()))
    o_ref[...] = (
        lax.dot_general(agg_ref[...], wl_ref[...], cdims,
                        preferred_element_type=jnp.float32)
        + lax.dot_general(xt_ref[...], wr_ref[...], cdims,
                          preferred_element_type=jnp.float32)
        + jnp.broadcast_to(b_ref[...], (N_GENES, _GTAIL))
    )


def _tc_gene_layer(xt, mn, wl, wr, b2d):
    main, agg = pl.pallas_call(
        _gene_layer_body,
        grid=(_GMAIN // _GBC,),
        in_specs=[
            pl.BlockSpec((N_GENES, N_CELLS), lambda c: (0, 0)),
            pl.BlockSpec((N_GENES, N_GENES), lambda c: (0, 0)),
            pl.BlockSpec((_GBC, N_CELLS), lambda c: (c, 0)),
            pl.BlockSpec((_GBC, N_CELLS), lambda c: (c, 0)),
            pl.BlockSpec((1, _GBC), lambda c: (0, c)),
        ],
        out_specs=[
            pl.BlockSpec((N_GENES, _GBC), lambda c: (0, c)),
            pl.BlockSpec((N_GENES, N_CELLS), lambda c: (0, 0)),
        ],
        out_shape=[
            jax.ShapeDtypeStruct((N_GENES, _GMAIN), jnp.float32),
            jax.ShapeDtypeStruct((N_GENES, N_CELLS), jnp.float32),
        ],
    )(xt, mn, wl, wr, b2d)
    tail = pl.pallas_call(
        _gene_tail_body,
        grid=(1,),
        in_specs=[
            pl.BlockSpec((N_GENES, N_CELLS), lambda c: (0, 0)),
            pl.BlockSpec((N_GENES, N_CELLS), lambda c: (0, 0)),
            pl.BlockSpec((_GTAIL, N_CELLS), lambda c: (_GMAIN // _GTAIL, 0)),
            pl.BlockSpec((_GTAIL, N_CELLS), lambda c: (_GMAIN // _GTAIL, 0)),
            pl.BlockSpec((1, _GTAIL), lambda c: (0, 0)),
        ],
        out_specs=pl.BlockSpec((N_GENES, _GTAIL), lambda c: (0, 0)),
        out_shape=jax.ShapeDtypeStruct((N_GENES, _GTAIL), jnp.float32),
    )(xt, agg, wl, wr, b2d[:, _GMAIN:])
    return jnp.concatenate([main, tail], axis=1)


def kernel(x, cell_graph, gene_graph, Wl_c, Wr_c, b_c, Wl_g, Wr_g, b_g):
    pad = _EPAD - E_CELL
    fill = jnp.full((pad,), N_CELLS, jnp.int32)
    srcp = jnp.concatenate([cell_graph[0], fill]).reshape(_NW, _CH, _CK)
    dstp = jnp.concatenate([cell_graph[1], fill]).reshape(_NW, _CH, _CK)
    zeros = jnp.zeros((_NPAD, 128), jnp.float32)
    b_c2 = b_c.reshape(1, N_GENES)
    b_g2 = b_g.reshape(1, N_CELLS)

    mn = _tc_gene_m(gene_graph)
    ones_tab = jnp.ones((N_CELLS + 1, N_GENES), jnp.float32)
    cnts = _make_sc_cell_agg()(ones_tab, srcp, dstp, zeros)

    h = x
    for _ in range(2):
        xe = jnp.concatenate([h, jnp.zeros((1, N_GENES), jnp.float32)])
        sums = _make_sc_cell_agg()(xe, srcp, dstp, zeros)
        h = _tc_cell_update(sums, cnts, h, Wl_c, Wr_c, b_c2)

    ht = h.T
    for _ in range(2):
        ht = _tc_gene_layer(ht, mn, Wl_g, Wr_g, b_g2)
    return ht
